# baseline (speedup 1.0000x reference)
import functools

import jax
import jax.numpy as jnp
from jax import lax
from jax.experimental import pallas as pl
from jax.experimental.pallas import tpu as pltpu
from jax.experimental.pallas import tpu_sc as plsc

B, K, L = 32, 1024, 576
LW = 32
NLANES = 16
NGROUPS = LW // NLANES
NUNITS = L // LW

_mesh = plsc.VectorSubcoreMesh(core_axis_name="c", subcore_axis_name="s")


@functools.partial(
    pl.kernel,
    out_type=jax.ShapeDtypeStruct((B, K, L), jnp.float32),
    mesh=_mesh,
    scratch_types=[pltpu.VMEM((K, LW), jnp.float32)],
    compiler_params=pltpu.CompilerParams(use_tc_tiling_on_sc=False),
)
def _argmax_onehot(x_hbm, out_hbm, buf):
    b = lax.axis_index("s") * 2 + lax.axis_index("c")

    for j in range(NUNITS):
        pltpu.sync_copy(x_hbm.at[b, :, pl.ds(j * LW, LW)], buf)

        for g in range(NGROUPS):
            cols = pl.ds(g * NLANES, NLANES)

            def pass1(k, m):
                return jnp.maximum(m, buf[k, cols])

            maxv = lax.fori_loop(
                0, K, pass1, jnp.full((NLANES,), -jnp.inf, jnp.float32)
            )

            def pass2(k, found):
                v = buf[k, cols]
                hit = jnp.where((v >= maxv) & (found == 0), 1, 0)
                buf[k, cols] = hit.astype(jnp.float32)
                return found | hit

            lax.fori_loop(0, K, pass2, jnp.zeros((NLANES,), jnp.int32))

        pltpu.sync_copy(buf, out_hbm.at[b, :, pl.ds(j * LW, LW)])


def kernel(x):
    return _argmax_onehot(x)


# speedup vs baseline: 1.7527x; 1.7527x over previous
import functools

import jax
import jax.numpy as jnp
from jax import lax
from jax.experimental import pallas as pl
from jax.experimental.pallas import tpu as pltpu
from jax.experimental.pallas import tpu_sc as plsc

B, K, L = 32, 1024, 576
KL = K * L
RPC = 64
CH = RPC * L
NCHUNK = K // RPC
NG = L // 16
NSCAT = 5

_mesh = plsc.VectorSubcoreMesh(core_axis_name="c", subcore_axis_name="s")


@functools.partial(
    pl.kernel,
    out_type=jax.ShapeDtypeStruct((B * KL,), jnp.float32),
    mesh=_mesh,
    scratch_types=[
        pltpu.VMEM((CH,), jnp.float32),
        pltpu.VMEM((CH,), jnp.float32),
        pltpu.VMEM((CH,), jnp.float32),
        pltpu.VMEM((L,), jnp.float32),
        pltpu.VMEM((L,), jnp.int32),
        pltpu.VMEM((NSCAT, 128), jnp.int32),
        pltpu.VMEM((128,), jnp.float32),
        pltpu.SemaphoreType.DMA,
        pltpu.SemaphoreType.DMA,
        pltpu.SemaphoreType.DMA,
        pltpu.SemaphoreType.DMA,
    ],
)
def _argmax_onehot(x_hbm, out_hbm, buf0, buf1, zbuf, m_ref, idx_ref,
                   fidx_ref, ones_ref, sem0, sem1, semz, semsc):
    b = lax.axis_index("s") * 2 + lax.axis_index("c")
    base = b * KL

    zv = jnp.zeros((16,), jnp.float32)

    def zset(i, _):
        zbuf[pl.ds(i * 16, 16)] = zv
        return 0

    lax.fori_loop(0, CH // 16, zset, 0, unroll=8)

    zhandles = [
        pltpu.async_copy(zbuf, out_hbm.at[pl.ds(base + c * CH, CH)], semz)
        for c in range(NCHUNK)
    ]

    bufs = (buf0, buf1)
    in_sems = (sem0, sem1)
    h_in = [
        pltpu.async_copy(x_hbm.at[pl.ds(base + c * CH, CH)], bufs[c],
                         in_sems[c])
        for c in range(2)
    ]

    ninf = jnp.full((16,), -jnp.inf, jnp.float32)
    iz = jnp.zeros((16,), jnp.int32)

    def initg(g, _):
        m_ref[pl.ds(g * 16, 16)] = ninf
        idx_ref[pl.ds(g * 16, 16)] = iz
        return 0

    lax.fori_loop(0, NG, initg, 0, unroll=4)

    for c in range(NCHUNK):
        h_in[c % 2].wait()
        buf = bufs[c % 2]
        rowbase = c * RPC

        def gbody(g, _, buf=buf, rowbase=rowbase):
            cols = pl.ds(g * 16, 16)
            m = m_ref[cols]
            ix = idx_ref[cols]

            def rbody(r, carry, buf=buf, g=g, rowbase=rowbase):
                m, ix = carry
                v = buf[pl.ds(r * L + g * 16, 16)]
                gt = v > m
                m2 = jnp.where(gt, v, m)
                rs = jnp.full((16,), rowbase + r, jnp.int32)
                ix2 = jnp.where(gt, rs, ix)
                return (m2, ix2)

            m, ix = lax.fori_loop(0, RPC, rbody, (m, ix), unroll=8)
            m_ref[cols] = m
            idx_ref[cols] = ix
            return 0

        lax.fori_loop(0, NG, gbody, 0)

        if c + 2 < NCHUNK:
            h_in[c % 2] = pltpu.async_copy(
                x_hbm.at[pl.ds(base + (c + 2) * CH, CH)], bufs[c % 2],
                in_sems[c % 2])

    onesv = jnp.full((16,), 1.0, jnp.float32)
    for t in range(8):
        ones_ref[pl.ds(t * 16, 16)] = onesv

    for g in range(NG):
        ix = idx_ref[pl.ds(g * 16, 16)]
        lvec = lax.iota(jnp.int32, 16) + g * 16
        f = base + ix * L + lvec
        fidx_ref[g * 16 // 128, pl.ds(g * 16 % 128, 16)] = f

    last = fidx_ref[4, pl.ds(48, 16)]
    dup = jnp.full((16,), last[15], jnp.int32)
    for t in range(4):
        fidx_ref[4, pl.ds(64 + t * 16, 16)] = dup

    for h in zhandles:
        h.wait()

    shandles = [
        pltpu.async_copy(ones_ref, out_hbm.at[fidx_ref.at[j]], semsc)
        for j in range(NSCAT)
    ]
    for h in shandles:
        h.wait()


def kernel(x):
    out = _argmax_onehot(x.reshape(-1))
    return out.reshape(B, K, L)


# speedup vs baseline: 1.7864x; 1.0192x over previous
import functools

import jax
import jax.numpy as jnp
from jax import lax
from jax.experimental import pallas as pl
from jax.experimental.pallas import tpu as pltpu
from jax.experimental.pallas import tpu_sc as plsc

B, K, L = 32, 1024, 576
KL = K * L
RPC = 64
CH = RPC * L
NCHUNK = K // RPC
NG = L // 16
NSCAT = 5

_mesh = plsc.VectorSubcoreMesh(core_axis_name="c", subcore_axis_name="s")


@functools.partial(
    pl.kernel,
    out_type=jax.ShapeDtypeStruct((B * KL,), jnp.float32),
    mesh=_mesh,
    scratch_types=[
        pltpu.VMEM((CH,), jnp.float32),
        pltpu.VMEM((CH,), jnp.float32),
        pltpu.VMEM((CH,), jnp.float32),
        pltpu.VMEM((L,), jnp.float32),
        pltpu.VMEM((L,), jnp.int32),
        pltpu.VMEM((NSCAT, 128), jnp.int32),
        pltpu.VMEM((128,), jnp.float32),
        pltpu.SemaphoreType.DMA,
        pltpu.SemaphoreType.DMA,
        pltpu.SemaphoreType.DMA,
        pltpu.SemaphoreType.DMA,
    ],
)
def _argmax_onehot(x_hbm, out_hbm, buf0, buf1, zbuf, m_ref, idx_ref,
                   fidx_ref, ones_ref, sem0, sem1, semz, semsc):
    b = lax.axis_index("s") * 2 + lax.axis_index("c")
    base = b * KL

    bufs = (buf0, buf1)
    in_sems = (sem0, sem1)
    h_in = [
        pltpu.async_copy(x_hbm.at[pl.ds(base + c * CH, CH)], bufs[c],
                         in_sems[c])
        for c in range(2)
    ]

    zv = jnp.zeros((16,), jnp.float32)

    def zset(i, _):
        zbuf[pl.ds(i * 16, 16)] = zv
        return 0

    lax.fori_loop(0, CH // 16, zset, 0, unroll=8)

    ninf = jnp.full((16,), -jnp.inf, jnp.float32)
    iz = jnp.zeros((16,), jnp.int32)

    def initg(g, _):
        m_ref[pl.ds(g * 16, 16)] = ninf
        idx_ref[pl.ds(g * 16, 16)] = iz
        return 0

    lax.fori_loop(0, NG, initg, 0, unroll=4)

    G2 = 4
    zhandles = []
    for c in range(NCHUNK):
        h_in[c % 2].wait()
        zhandles.append(
            pltpu.async_copy(zbuf, out_hbm.at[pl.ds(base + c * CH, CH)],
                             semz))
        buf = bufs[c % 2]
        rowbase = c * RPC

        def gbody(gq, _, buf=buf, rowbase=rowbase):
            g0 = gq * G2
            cols = [pl.ds((g0 + i) * 16, 16) for i in range(G2)]
            carry = []
            for i in range(G2):
                carry.append(m_ref[cols[i]])
                carry.append(idx_ref[cols[i]])

            def rbody(r, carry, buf=buf, g0=g0, rowbase=rowbase):
                out = []
                rs = jnp.full((16,), rowbase + r, jnp.int32)
                off = r * L + g0 * 16
                for i in range(G2):
                    m, ix = carry[2 * i], carry[2 * i + 1]
                    v = buf[pl.ds(off + i * 16, 16)]
                    gt = v > m
                    out.append(jnp.where(gt, v, m))
                    out.append(jnp.where(gt, rs, ix))
                return tuple(out)

            carry = lax.fori_loop(0, RPC, rbody, tuple(carry), unroll=4)
            for i in range(G2):
                m_ref[cols[i]] = carry[2 * i]
                idx_ref[cols[i]] = carry[2 * i + 1]
            return 0

        lax.fori_loop(0, NG // G2, gbody, 0)

        if c + 2 < NCHUNK:
            h_in[c % 2] = pltpu.async_copy(
                x_hbm.at[pl.ds(base + (c + 2) * CH, CH)], bufs[c % 2],
                in_sems[c % 2])

    onesv = jnp.full((16,), 1.0, jnp.float32)
    for t in range(8):
        ones_ref[pl.ds(t * 16, 16)] = onesv

    for g in range(NG):
        ix = idx_ref[pl.ds(g * 16, 16)]
        lvec = lax.iota(jnp.int32, 16) + g * 16
        f = base + ix * L + lvec
        fidx_ref[g * 16 // 128, pl.ds(g * 16 % 128, 16)] = f

    last = fidx_ref[4, pl.ds(48, 16)]
    dup = jnp.full((16,), last[15], jnp.int32)
    for t in range(4):
        fidx_ref[4, pl.ds(64 + t * 16, 16)] = dup

    for h in zhandles:
        h.wait()

    shandles = [
        pltpu.async_copy(ones_ref, out_hbm.at[fidx_ref.at[j]], semsc)
        for j in range(NSCAT)
    ]
    for h in shandles:
        h.wait()


def kernel(x):
    out = _argmax_onehot(x.reshape(-1))
    return out.reshape(B, K, L)


# speedup vs baseline: 3.0994x; 1.7350x over previous
import functools

import jax
import jax.numpy as jnp
from jax import lax
from jax.experimental import pallas as pl
from jax.experimental.pallas import tpu as pltpu
from jax.experimental.pallas import tpu_sc as plsc

B, K, L = 32, 1024, 576
RPC = 64
NCHUNK = K // RPC
NG = L // 16
G2 = 4

_mesh = plsc.VectorSubcoreMesh(core_axis_name="c", subcore_axis_name="s")


@functools.partial(
    pl.kernel,
    out_type=jax.ShapeDtypeStruct((B, K, L), jnp.float32),
    mesh=_mesh,
    scratch_types=[
        pltpu.VMEM((RPC, L), jnp.float32),
        pltpu.VMEM((RPC, L), jnp.float32),
        pltpu.VMEM((L,), jnp.float32),
        pltpu.VMEM((L,), jnp.int32),
        pltpu.SemaphoreType.DMA,
        pltpu.SemaphoreType.DMA,
        pltpu.SemaphoreType.DMA,
        pltpu.SemaphoreType.DMA,
    ],
)
def _argmax_onehot(x_hbm, out_hbm, buf0, buf1, m_ref, idx_ref,
                   si0, si1, so0, so1):
    b = lax.axis_index("s") * 2 + lax.axis_index("c")
    bufs = (buf0, buf1)
    in_sems = (si0, si1)
    out_sems = (so0, so1)

    h_in = [
        pltpu.async_copy(x_hbm.at[b, pl.ds(c * RPC, RPC), :], bufs[c],
                         in_sems[c])
        for c in range(2)
    ]

    ninf = jnp.full((16,), -jnp.inf, jnp.float32)
    iz = jnp.zeros((16,), jnp.int32)

    def initg(g, _):
        m_ref[pl.ds(g * 16, 16)] = ninf
        idx_ref[pl.ds(g * 16, 16)] = iz
        return 0

    lax.fori_loop(0, NG, initg, 0, unroll=4)

    for c in range(NCHUNK):
        h_in[c % 2].wait()
        buf = bufs[c % 2]
        rowbase = c * RPC

        def gbody(gq, _, buf=buf, rowbase=rowbase):
            g0 = gq * G2
            cols = [pl.ds((g0 + i) * 16, 16) for i in range(G2)]
            carry = []
            for i in range(G2):
                carry.append(m_ref[cols[i]])
                carry.append(idx_ref[cols[i]])

            def rbody(r, carry, buf=buf, g0=g0, rowbase=rowbase):
                out = []
                rs = jnp.full((16,), rowbase + r, jnp.int32)
                for i in range(G2):
                    m, ix = carry[2 * i], carry[2 * i + 1]
                    v = buf[r, pl.ds((g0 + i) * 16, 16)]
                    gt = v > m
                    out.append(jnp.where(gt, v, m))
                    out.append(jnp.where(gt, rs, ix))
                return tuple(out)

            carry = lax.fori_loop(0, RPC, rbody, tuple(carry), unroll=4)
            for i in range(G2):
                m_ref[cols[i]] = carry[2 * i]
                idx_ref[cols[i]] = carry[2 * i + 1]
            return 0

        lax.fori_loop(0, NG // G2, gbody, 0)

        if c + 2 < NCHUNK:
            h_in[c % 2] = pltpu.async_copy(
                x_hbm.at[b, pl.ds((c + 2) * RPC, RPC), :], bufs[c % 2],
                in_sems[c % 2])

    onev = jnp.full((16,), 1.0, jnp.float32)
    zerov = jnp.zeros((16,), jnp.float32)
    h_out = [None, None]
    for c in range(NCHUNK):
        buf = bufs[c % 2]
        if c >= 2:
            h_out[c % 2].wait()
        rowbase = c * RPC

        def g2body(gq, _, buf=buf, rowbase=rowbase):
            g0 = gq * G2
            ixs = [idx_ref[pl.ds((g0 + i) * 16, 16)] for i in range(G2)]

            def r2body(r, _, buf=buf, g0=g0, rowbase=rowbase, ixs=ixs):
                rs = jnp.full((16,), rowbase + r, jnp.int32)
                for i in range(G2):
                    buf[r, pl.ds((g0 + i) * 16, 16)] = jnp.where(
                        ixs[i] == rs, onev, zerov)
                return 0

            lax.fori_loop(0, RPC, r2body, 0, unroll=4)
            return 0

        lax.fori_loop(0, NG // G2, g2body, 0)

        h_out[c % 2] = pltpu.async_copy(
            buf, out_hbm.at[b, pl.ds(c * RPC, RPC), :], out_sems[c % 2])

    h_out[0].wait()
    h_out[1].wait()


def kernel(x):
    return _argmax_onehot(x)
